# baseline (speedup 1.0000x reference)
import functools

import numpy as np
import jax
import jax.numpy as jnp
from jax import lax
from jax.experimental import pallas as pl
from jax.experimental.pallas import tpu as pltpu
from jax.experimental.pallas import tpu_sc as plsc

_VOCAB = 1000000
_EMBED = 64
_SEQ = 200
_BATCH = 4096

_NW = 32
_JB = _BATCH // 128
_UNITS = _SEQ * _JB
_PW = _UNITS // _NW
_GROUPS = _PW // 8
_L = 16
_DT = _EMBED // 8
_GS = 65


def _make_pe():
    pos = np.arange(_SEQ, dtype=np.float32)[:, None]
    div = np.exp(np.arange(0, _EMBED, 2, dtype=np.float32)
                 * -(np.log(10000.0) / _EMBED))
    pe = np.zeros((_SEQ, _EMBED), np.float32)
    pe[:, 0::2] = np.sin(pos * div)
    pe[:, 1::2] = np.cos(pos * div)
    return pe


_PES = np.repeat(_make_pe()[:, :, None], _L, axis=2)


def _sc_embed(seq_lin, table, pes):
    mesh = plsc.VectorSubcoreMesh(core_axis_name="c", subcore_axis_name="s")

    @functools.partial(
        pl.kernel,
        mesh=mesh,
        out_type=jax.ShapeDtypeStruct((_SEQ, _DT, _JB, 8, 128), jnp.float32),
        scratch_types=[
            pltpu.VMEM((2, 8, 128), jnp.int32),
            pltpu.VMEM((2, 512, _EMBED), jnp.float32),
            pltpu.VMEM((128, _GS), jnp.float32),
            pltpu.VMEM((2, _DT, 8, 128), jnp.float32),
            pltpu.VMEM((_EMBED, _L), jnp.float32),
            pltpu.SemaphoreType.DMA,
            pltpu.SemaphoreType.DMA,
        ],
        compiler_params=pltpu.CompilerParams(use_tc_tiling_on_sc=False,
                                             needs_layout_passes=False),
    )
    def k(seq_hbm, tab_hbm, pes_hbm, out_hbm, idx_v, g, g65, st, pes_v, gsem,
          wsem):
        wid = lax.axis_index("s") * 2 + lax.axis_index("c")
        base_r = wid * _PW
        tok_vecs = [lax.iota(jnp.int32, _L) + (h * _L) for h in range(8)]

        def unit_compute(b, u, j):
            def stage_body(r, carry2):
                for kk in range(_EMBED // _L):
                    sl = pl.ds(kk * _L, _L)
                    g65[r, sl] = g[b, u * 128 + r, sl]
                return carry2

            lax.fori_loop(0, 128, stage_body, 0, unroll=8)

            def d_body(d, carry2):
                pe_val = pes_v[d]
                d_splat = jnp.full((_L,), d, jnp.int32)
                vs = [plsc.load_gather(g65, [tok_vecs[h], d_splat])
                      for h in range(8)]
                vs = [v + pe_val for v in vs]
                for h in range(8):
                    st[u % 2, d // 8, lax.rem(d, 8), pl.ds(h * _L, _L)] = vs[h]
                return carry2

            lax.fori_loop(0, _EMBED, d_body, 0, unroll=2)

        def idx_copy(gi, b):
            gic = jnp.minimum(gi, _GROUPS - 1)
            gr8 = pl.multiple_of(base_r + gic * 8, 8)
            pltpu.sync_copy(seq_hbm.at[pl.ds(gr8, 8)], idx_v.at[b])

        def fire(b, half, gbuf):
            for u in range(4):
                pltpu.async_copy(tab_hbm.at[idx_v.at[b, half * 4 + u]],
                                 g.at[gbuf, pl.ds(u * 128, 128)], gsem)

        def drain(gbuf):
            for u in range(4):
                pltpu.make_async_copy(tab_hbm.at[idx_v.at[0, u]],
                                      g.at[gbuf, pl.ds(u * 128, 128)],
                                      gsem).wait()

        def half_compute(gbuf, s, j0):
            wcs = {}
            for u in range(4):
                if u - 2 in wcs:
                    wcs.pop(u - 2).wait()
                unit_compute(gbuf, u, j0 + u)
                wcs[u] = pltpu.async_copy(st.at[u % 2],
                                          out_hbm.at[s, :, j0 + u], wsem)
            for wc in wcs.values():
                wc.wait()

        idx_copy(0, 0)
        fire(0, 0, 0)

        def group_body(gi, carry):
            b = lax.rem(gi, 2)
            gr = base_r + gi * 8
            s = gr // _JB
            j0 = lax.rem(gr, _JB)
            fire(b, 1, 1)
            idx_copy(gi + 1, 1 - b)
            pltpu.sync_copy(pes_hbm.at[s], pes_v)
            drain(0)
            half_compute(0, s, j0)
            fire(1 - b, 0, 0)
            drain(1)
            half_compute(1, s, j0 + 4)
            return carry

        lax.fori_loop(0, _GROUPS, group_body, 0)
        drain(0)

    return k(seq_lin, table, pes)


@jax.jit
def kernel(sequence, table):
    seq_lin = jnp.transpose(sequence).reshape(_UNITS, 128).astype(jnp.int32)
    pes = jnp.asarray(_PES)
    lin5 = _sc_embed(seq_lin, table, pes)
    return lin5.transpose(2, 4, 0, 1, 3).reshape(_BATCH, _SEQ, _EMBED)


# speedup vs baseline: 1.0144x; 1.0144x over previous
import functools

import numpy as np
import jax
import jax.numpy as jnp
from jax import lax
from jax.experimental import pallas as pl
from jax.experimental.pallas import tpu as pltpu
from jax.experimental.pallas import tpu_sc as plsc

_VOCAB = 1000000
_EMBED = 64
_SEQ = 200
_BATCH = 4096

_NW = 32
_JB = _BATCH // 128
_UNITS = _SEQ * _JB
_PW = _UNITS // _NW
_GROUPS = _PW // 8
_L = 16
_DT = _EMBED // 8
_GS = 65


def _make_pe():
    pos = np.arange(_SEQ, dtype=np.float32)[:, None]
    div = np.exp(np.arange(0, _EMBED, 2, dtype=np.float32)
                 * -(np.log(10000.0) / _EMBED))
    pe = np.zeros((_SEQ, _EMBED), np.float32)
    pe[:, 0::2] = np.sin(pos * div)
    pe[:, 1::2] = np.cos(pos * div)
    return pe


_PES = np.repeat(_make_pe()[:, :, None], _L, axis=2)


def _sc_embed(seq_lin, table, pes):
    mesh = plsc.VectorSubcoreMesh(core_axis_name="c", subcore_axis_name="s")

    @functools.partial(
        pl.kernel,
        mesh=mesh,
        out_type=jax.ShapeDtypeStruct((_SEQ, _DT, _JB, 8, 128), jnp.float32),
        scratch_types=[
            pltpu.VMEM((2, 8, 128), jnp.int32),
            pltpu.VMEM((2, 512, _EMBED), jnp.float32),
            pltpu.VMEM((128, _GS), jnp.float32),
            pltpu.VMEM((2, _DT, 8, 128), jnp.float32),
            pltpu.VMEM((_EMBED, _L), jnp.float32),
            pltpu.SemaphoreType.DMA,
            pltpu.SemaphoreType.DMA,
            pltpu.SemaphoreType.DMA,
            pltpu.SemaphoreType.DMA,
        ],
        compiler_params=pltpu.CompilerParams(use_tc_tiling_on_sc=False,
                                             needs_layout_passes=False),
    )
    def k(seq_hbm, tab_hbm, pes_hbm, out_hbm, idx_v, g, g65, st, pes_v, gsem,
          wsem, isem, psem):
        wid = lax.axis_index("s") * 2 + lax.axis_index("c")
        base_r = wid * _PW
        tok_vecs = [lax.iota(jnp.int32, _L) + (h * _L) for h in range(8)]

        def unit_compute(b, u, j):
            def stage_body(r, carry2):
                for kk in range(_EMBED // _L):
                    sl = pl.ds(kk * _L, _L)
                    g65[r, sl] = g[b, u * 128 + r, sl]
                return carry2

            lax.fori_loop(0, 128, stage_body, 0, unroll=8)

            def d_body(d, carry2):
                pe_val = pes_v[d]
                d_splat = jnp.full((_L,), d, jnp.int32)
                vs = [plsc.load_gather(g65, [tok_vecs[h], d_splat])
                      for h in range(8)]
                vs = [v + pe_val for v in vs]
                for h in range(8):
                    st[u % 2, d // 8, lax.rem(d, 8), pl.ds(h * _L, _L)] = vs[h]
                return carry2

            lax.fori_loop(0, _EMBED, d_body, 0, unroll=2)

        def idx_copy(gi, b):
            gic = jnp.minimum(gi, _GROUPS - 1)
            gr8 = pl.multiple_of(base_r + gic * 8, 8)
            return pltpu.async_copy(seq_hbm.at[pl.ds(gr8, 8)], idx_v.at[b],
                                    isem)

        def fire(b, half, gbuf):
            for u in range(4):
                pltpu.async_copy(tab_hbm.at[idx_v.at[b, half * 4 + u]],
                                 g.at[gbuf, pl.ds(u * 128, 128)], gsem)

        def drain(gbuf):
            for u in range(4):
                pltpu.make_async_copy(tab_hbm.at[idx_v.at[0, u]],
                                      g.at[gbuf, pl.ds(u * 128, 128)],
                                      gsem).wait()

        def half_compute(gbuf, s, j0):
            wcs = {}
            for u in range(4):
                if u - 2 in wcs:
                    wcs.pop(u - 2).wait()
                unit_compute(gbuf, u, j0 + u)
                wcs[u] = pltpu.async_copy(st.at[u % 2],
                                          out_hbm.at[s, :, j0 + u], wsem)
            for wc in wcs.values():
                wc.wait()

        idx_copy(0, 0).wait()
        fire(0, 0, 0)

        def group_body(gi, carry):
            b = lax.rem(gi, 2)
            gr = base_r + gi * 8
            s = gr // _JB
            j0 = lax.rem(gr, _JB)
            fire(b, 1, 1)
            icp = idx_copy(gi + 1, 1 - b)
            pcp = pltpu.async_copy(pes_hbm.at[s], pes_v, psem)
            drain(0)
            pcp.wait()
            half_compute(0, s, j0)
            icp.wait()
            fire(1 - b, 0, 0)
            drain(1)
            half_compute(1, s, j0 + 4)
            return carry

        lax.fori_loop(0, _GROUPS, group_body, 0)
        drain(0)

    return k(seq_lin, table, pes)


@jax.jit
def kernel(sequence, table):
    seq_lin = jnp.transpose(sequence).reshape(_UNITS, 128).astype(jnp.int32)
    pes = jnp.asarray(_PES)
    lin5 = _sc_embed(seq_lin, table, pes)
    return lin5.transpose(2, 4, 0, 1, 3).reshape(_BATCH, _SEQ, _EMBED)
